# baseline (speedup 1.0000x reference)
import functools

import jax
import jax.numpy as jnp
from jax import lax
from jax.experimental import pallas as pl
from jax.experimental.pallas import tpu as pltpu
from jax.experimental.pallas import tpu_sc as plsc

_B = 4096
_D = 768
_H = 512


def _make_sc_gather(b):
    info = plsc.get_sparse_core_info()
    nc, ns = info.num_cores, info.num_subcores
    nw = nc * ns
    b_per_w = b // nw
    mesh = plsc.VectorSubcoreMesh(core_axis_name="c", subcore_axis_name="s")

    @functools.partial(
        pl.kernel,
        mesh=mesh,
        out_type=[
            jax.ShapeDtypeStruct((b, _D), jnp.float32),
            jax.ShapeDtypeStruct((b, _D), jnp.float32),
        ],
        scratch_types=[
            pltpu.VMEM((b_per_w,), jnp.int32),
            pltpu.VMEM((b_per_w, _D), jnp.float32),
            pltpu.SemaphoreType.DMA,
        ],
    )
    def gather_k(item_idx, user_idx, item_table, user_table,
                 item_out, user_out, idx_v, rows_v, sem):
        wid = lax.axis_index("s") * nc + lax.axis_index("c")
        base = wid * b_per_w
        pltpu.sync_copy(item_idx.at[pl.ds(base, b_per_w)], idx_v)
        pltpu.async_copy(item_table.at[idx_v], rows_v, sem).wait()
        pltpu.sync_copy(rows_v, item_out.at[pl.ds(base, b_per_w)])
        pltpu.sync_copy(user_idx.at[pl.ds(base, b_per_w)], idx_v)
        pltpu.async_copy(user_table.at[idx_v], rows_v, sem).wait()
        pltpu.sync_copy(rows_v, user_out.at[pl.ds(base, b_per_w)])

    return gather_k


_sc_gather = _make_sc_gather(_B)


def _mlp_body(item_lo, item_hi, user_lo, user_hi,
              wb2_lo, wb2_hi, bb2_ref, wa2_lo, wa2_hi, ba2_ref,
              wl1_ref, bl1_ref, out_ref):
    def enc(lo, hi, w_lo, w_hi, bias):
        acc = jnp.dot(lo[...].astype(jnp.bfloat16), w_lo[...],
                      preferred_element_type=jnp.float32)
        acc += jnp.dot(hi[...].astype(jnp.bfloat16), w_hi[...],
                       preferred_element_type=jnp.float32)
        return jnp.maximum(acc + bias[...], 0.0)

    ienc = enc(item_lo, item_hi, wb2_lo, wb2_hi, bb2_ref)
    uenc = enc(user_lo, user_hi, wa2_lo, wa2_hi, ba2_ref)
    prod = (ienc * uenc).astype(jnp.bfloat16)
    out = lax.dot_general(wl1_ref[...], prod, (((1,), (1,)), ((), ())),
                          preferred_element_type=jnp.float32)
    out_ref[...] = jnp.maximum(out + bl1_ref[...], 0.0)[None]


_BT = 1024


def _mlp(item_emb, user_emb, W_b2, b_b2, W_a2, b_a2, W_l1t, b_l1):
    b = item_emb.shape[0]
    g = b // _BT
    hd = _D // 2
    emb_lo = pl.BlockSpec((_BT, hd), lambda i: (i, 0))
    emb_hi = pl.BlockSpec((_BT, hd), lambda i: (i, 1))
    w_lo = pl.BlockSpec((hd, _H), lambda i: (0, 0))
    w_hi = pl.BlockSpec((hd, _H), lambda i: (1, 0))
    out = pl.pallas_call(
        _mlp_body,
        grid=(g,),
        in_specs=[
            emb_lo, emb_hi, emb_lo, emb_hi,
            w_lo, w_hi,
            pl.BlockSpec((1, _H), lambda i: (0, 0)),
            w_lo, w_hi,
            pl.BlockSpec((1, _H), lambda i: (0, 0)),
            pl.BlockSpec((1, _H), lambda i: (0, 0)),
            pl.BlockSpec((1, 1), lambda i: (0, 0)),
        ],
        out_specs=pl.BlockSpec((1, 1, _BT), lambda i: (i, 0, 0)),
        out_shape=jax.ShapeDtypeStruct((g, 1, _BT), jnp.float32),
    )(item_emb, item_emb, user_emb, user_emb,
      W_b2, W_b2, b_b2, W_a2, W_a2, b_a2, W_l1t, b_l1)
    return out.reshape(b, 1)


def kernel(item_vec, user_vec, item_table, user_table,
           W_b2, b_b2, W_a2, b_a2, W_l1, b_l1):
    item_idx = item_vec.astype(jnp.int32)
    user_idx = user_vec.astype(jnp.int32)
    wb2 = W_b2.astype(jnp.bfloat16)
    wa2 = W_a2.astype(jnp.bfloat16)
    wl1t = W_l1.reshape(1, _H).astype(jnp.bfloat16)
    bb2 = b_b2.reshape(1, _H)
    ba2 = b_a2.reshape(1, _H)
    bl1 = b_l1.reshape(1, 1)
    item_emb, user_emb = _sc_gather(item_idx, user_idx,
                                    item_table, user_table)
    return _mlp(item_emb, user_emb, wb2, bb2, wa2, ba2, wl1t, bl1)


# speedup vs baseline: 1.0202x; 1.0202x over previous
import functools

import jax
import jax.numpy as jnp
from jax import lax
from jax.experimental import pallas as pl
from jax.experimental.pallas import tpu as pltpu
from jax.experimental.pallas import tpu_sc as plsc

_B = 4096
_D = 768
_H = 512


def _make_sc_gather(b):
    info = plsc.get_sparse_core_info()
    nc, ns = info.num_cores, info.num_subcores
    nw = nc * ns
    b_per_w = b // nw
    mesh = plsc.VectorSubcoreMesh(core_axis_name="c", subcore_axis_name="s")

    @functools.partial(
        pl.kernel,
        mesh=mesh,
        out_type=[
            jax.ShapeDtypeStruct((b, _D), jnp.float32),
            jax.ShapeDtypeStruct((b, _D), jnp.float32),
        ],
        scratch_types=[
            pltpu.VMEM((b_per_w,), jnp.int32),
            pltpu.VMEM((b_per_w, _D), jnp.float32),
            pltpu.SemaphoreType.DMA,
        ],
    )
    def gather_k(item_idx, user_idx, item_table, user_table,
                 item_out, user_out, idx_v, rows_v, sem):
        wid = lax.axis_index("s") * nc + lax.axis_index("c")
        base = wid * b_per_w
        pltpu.sync_copy(item_idx.at[pl.ds(base, b_per_w)], idx_v)
        pltpu.async_copy(item_table.at[idx_v], rows_v, sem).wait()
        pltpu.sync_copy(rows_v, item_out.at[pl.ds(base, b_per_w)])
        pltpu.sync_copy(user_idx.at[pl.ds(base, b_per_w)], idx_v)
        pltpu.async_copy(user_table.at[idx_v], rows_v, sem).wait()
        pltpu.sync_copy(rows_v, user_out.at[pl.ds(base, b_per_w)])

    return gather_k


_sc_gather = _make_sc_gather(_B)


def _mlp_body(item_ref, user_ref, wb2_ref, bb2_ref, wa2_ref, ba2_ref,
              wl1_ref, bl1_ref, out_ref):
    item_b = item_ref[...].astype(jnp.bfloat16)
    user_b = user_ref[...].astype(jnp.bfloat16)
    ienc = jnp.maximum(
        jnp.dot(item_b, wb2_ref[...],
                preferred_element_type=jnp.float32) + bb2_ref[...], 0.0)
    uenc = jnp.maximum(
        jnp.dot(user_b, wa2_ref[...],
                preferred_element_type=jnp.float32) + ba2_ref[...], 0.0)
    prod = (ienc * uenc).astype(jnp.bfloat16)
    out = lax.dot_general(wl1_ref[...], prod, (((1,), (1,)), ((), ())),
                          preferred_element_type=jnp.float32)
    out_ref[...] = jnp.maximum(out + bl1_ref[...], 0.0)[None]


_BT = 1024


def _mlp(item_emb, user_emb, W_b2, b_b2, W_a2, b_a2, W_l1t, b_l1):
    b = item_emb.shape[0]
    g = b // _BT
    out = pl.pallas_call(
        _mlp_body,
        grid=(g,),
        in_specs=[
            pl.BlockSpec((_BT, _D), lambda i: (i, 0)),
            pl.BlockSpec((_BT, _D), lambda i: (i, 0)),
            pl.BlockSpec((_D, _H), lambda i: (0, 0)),
            pl.BlockSpec((1, _H), lambda i: (0, 0)),
            pl.BlockSpec((_D, _H), lambda i: (0, 0)),
            pl.BlockSpec((1, _H), lambda i: (0, 0)),
            pl.BlockSpec((1, _H), lambda i: (0, 0)),
            pl.BlockSpec((1, 1), lambda i: (0, 0)),
        ],
        out_specs=pl.BlockSpec((1, 1, _BT), lambda i: (i, 0, 0)),
        out_shape=jax.ShapeDtypeStruct((g, 1, _BT), jnp.float32),
    )(item_emb, user_emb, W_b2, b_b2, W_a2, b_a2, W_l1t, b_l1)
    return out.reshape(b, 1)


def kernel(item_vec, user_vec, item_table, user_table,
           W_b2, b_b2, W_a2, b_a2, W_l1, b_l1):
    item_idx = item_vec.astype(jnp.int32)
    user_idx = user_vec.astype(jnp.int32)
    wb2 = W_b2.astype(jnp.bfloat16)
    wa2 = W_a2.astype(jnp.bfloat16)
    wl1t = W_l1.reshape(1, _H).astype(jnp.bfloat16)
    bb2 = b_b2.reshape(1, _H)
    ba2 = b_a2.reshape(1, _H)
    bl1 = b_l1.reshape(1, 1)
    item_emb, user_emb = _sc_gather(item_idx, user_idx,
                                    item_table, user_table)
    return _mlp(item_emb, user_emb, wb2, bb2, wa2, ba2, wl1t, bl1)
